# baseline (speedup 1.0000x reference)
import functools

import jax
import jax.numpy as jnp
from jax import lax
from jax.experimental import pallas as pl
from jax.experimental.pallas import tpu as pltpu
from jax.experimental.pallas import tpu_sc as plsc

L = 16
NC = 2
NS = 16
NW = NC * NS
B = 8
HW = 224 * 224
CHUNK = HW // NW
NVEC = CHUNK // L
NQ = 5


def _bce(x, t):
    e = jnp.exp(-jnp.abs(x))
    r = e / (e + 2.0)
    s = r * r
    p = jnp.float32(1.0 / 11.0)
    for c in (1.0 / 9.0, 1.0 / 7.0, 1.0 / 5.0, 1.0 / 3.0, 1.0):
        p = p * s + jnp.float32(c)
    return jnp.maximum(x, 0.0) - x * t + 2.0 * r * p


def _sc_body(pred_hbm, target_hbm, out_hbm, buf, stage):
    wid = lax.axis_index("s") * NC + lax.axis_index("c")
    base = wid * CHUNK
    zero = jnp.zeros((L,), jnp.float32)

    for s in range(B):
        for c in range(3):
            off = (s * 3 + c) * HW + base
            pltpu.sync_copy(pred_hbm.at[pl.ds(off, CHUNK)], buf.at[c])
            pltpu.sync_copy(target_hbm.at[pl.ds(off, CHUNK)], buf.at[3 + c])

        def step(i, carry):
            acc_t2, acc_c, acc_ti, acc_w, acc_wc = carry
            o = i * L
            p0 = buf[0, pl.ds(o, L)]
            p1 = buf[1, pl.ds(o, L)]
            p2 = buf[2, pl.ds(o, L)]
            t0 = buf[3, pl.ds(o, L)]
            t1 = buf[4, pl.ds(o, L)]
            t2 = buf[5, pl.ds(o, L)]
            tc = t2 - t2 * t0
            roi = 1.0 - tc * (1.0 - t1)
            w = jnp.where(roi != 0.0, 1.0, 0.0)
            acc_t2 = acc_t2 + t2
            acc_c = acc_c + _bce(p2, tc)
            acc_ti = acc_ti + _bce(p0, t0)
            bw = _bce(p1, t1)
            acc_w = acc_w + bw * w
            acc_wc = acc_wc + w
            return acc_t2, acc_c, acc_ti, acc_w, acc_wc

        acc = lax.fori_loop(0, NVEC, step, (zero, zero, zero, zero, zero))
        for q in range(NQ):
            stage[q * B + s] = acc[q]

    pltpu.sync_copy(stage, out_hbm.at[wid])


def _combine_body(part_ref, o_ref):
    x = part_ref[...]
    z = jnp.sum(jnp.sum(x, axis=0), axis=-1)
    t2 = lax.slice(z, (0,), (B,))
    cs = lax.slice(z, (B,), (2 * B,))
    ti = lax.slice(z, (2 * B,), (3 * B,))
    wv = lax.slice(z, (3 * B,), (4 * B,))
    wc = lax.slice(z, (4 * B,), (5 * B,))
    sel = t2 != 0.0
    zero8 = jnp.zeros((B,), jnp.float32)
    cnum = jnp.sum(jnp.where(sel, cs, zero8))
    cden = jnp.sum(jnp.where(sel, jnp.float32(HW), zero8))
    centroid = jnp.where(cden > 0.0, cnum / jnp.maximum(cden, 1.0), 0.0)
    tissue = jnp.sum(ti) / jnp.float32(B * HW)
    wden = jnp.sum(wc)
    whole = jnp.where(wden > 0.0, jnp.sum(wv) / jnp.maximum(wden, 1.0), 0.0)
    o_ref[...] = jnp.reshape(whole + centroid + tissue, (1, 1))


@jax.jit
def kernel(pred, target):
    pf = pred.reshape(-1)
    tf = target.reshape(-1)
    mesh = plsc.VectorSubcoreMesh(core_axis_name="c", subcore_axis_name="s", num_cores=NC, num_subcores=NS)
    partials = pl.kernel(
        _sc_body,
        out_type=jax.ShapeDtypeStruct((NW, NQ * B, L), jnp.float32),
        mesh=mesh,
        scratch_types=[
            pltpu.VMEM((6, CHUNK), jnp.float32),
            pltpu.VMEM((NQ * B, L), jnp.float32),
        ],
        compiler_params=pltpu.CompilerParams(use_tc_tiling_on_sc=False),
    )(pf, tf)
    out = pl.pallas_call(
        _combine_body,
        out_shape=jax.ShapeDtypeStruct((1, 1), jnp.float32),
    )(partials)
    return out[0, 0]


# speedup vs baseline: 1.5414x; 1.5414x over previous
import functools

import jax
import jax.numpy as jnp
from jax import lax
from jax.experimental import pallas as pl
from jax.experimental.pallas import tpu as pltpu
from jax.experimental.pallas import tpu_sc as plsc

L = 16
NC = 2
NS = 16
NW = NC * NS
B = 8
HW = 224 * 224
CHUNK = HW // NW
NVEC = CHUNK // L
NQ = 5


def _bce(x, t):
    e = jnp.exp(-jnp.abs(x))
    r = e / (e + 2.0)
    s = r * r
    p = jnp.float32(1.0 / 11.0)
    for c in (1.0 / 9.0, 1.0 / 7.0, 1.0 / 5.0, 1.0 / 3.0, 1.0):
        p = p * s + jnp.float32(c)
    return jnp.maximum(x, 0.0) - x * t + 2.0 * r * p


PREFETCH = 2


def _sc_body(pred_hbm, target_hbm, out_hbm, buf, stage, *sems):
    wid = lax.axis_index("s") * NC + lax.axis_index("c")
    base = wid * CHUNK
    zero = jnp.zeros((L,), jnp.float32)

    def issue(s):
        handles = []
        for c in range(3):
            off = (s * 3 + c) * HW + base
            handles.append(pltpu.async_copy(
                pred_hbm.at[pl.ds(off, CHUNK)], buf.at[s, c], sems[s]))
            handles.append(pltpu.async_copy(
                target_hbm.at[pl.ds(off, CHUNK)], buf.at[s, 3 + c], sems[s]))
        return handles

    inflight = {s: issue(s) for s in range(PREFETCH)}

    for s in range(B):
        for h in inflight.pop(s):
            h.wait()
        if s + PREFETCH < B:
            inflight[s + PREFETCH] = issue(s + PREFETCH)

        def step(i, carry):
            acc_t2, acc_c, acc_ti, acc_w, acc_wc = carry
            o = i * L
            p0 = buf[s, 0, pl.ds(o, L)]
            p1 = buf[s, 1, pl.ds(o, L)]
            p2 = buf[s, 2, pl.ds(o, L)]
            t0 = buf[s, 3, pl.ds(o, L)]
            t1 = buf[s, 4, pl.ds(o, L)]
            t2 = buf[s, 5, pl.ds(o, L)]
            tc = t2 - t2 * t0
            roi = 1.0 - tc * (1.0 - t1)
            w = jnp.where(roi != 0.0, 1.0, 0.0)
            acc_t2 = acc_t2 + t2
            acc_c = acc_c + _bce(p2, tc)
            acc_ti = acc_ti + _bce(p0, t0)
            bw = _bce(p1, t1)
            acc_w = acc_w + bw * w
            acc_wc = acc_wc + w
            return acc_t2, acc_c, acc_ti, acc_w, acc_wc

        acc = lax.fori_loop(0, NVEC, step, (zero, zero, zero, zero, zero))
        for q in range(NQ):
            stage[q * B + s] = acc[q]

    pltpu.sync_copy(stage, out_hbm.at[wid])


def _combine_body(part_ref, o_ref):
    x = part_ref[...]
    z = jnp.sum(jnp.sum(x, axis=0), axis=-1)
    t2 = lax.slice(z, (0,), (B,))
    cs = lax.slice(z, (B,), (2 * B,))
    ti = lax.slice(z, (2 * B,), (3 * B,))
    wv = lax.slice(z, (3 * B,), (4 * B,))
    wc = lax.slice(z, (4 * B,), (5 * B,))
    sel = t2 != 0.0
    zero8 = jnp.zeros((B,), jnp.float32)
    cnum = jnp.sum(jnp.where(sel, cs, zero8))
    cden = jnp.sum(jnp.where(sel, jnp.float32(HW), zero8))
    centroid = jnp.where(cden > 0.0, cnum / jnp.maximum(cden, 1.0), 0.0)
    tissue = jnp.sum(ti) / jnp.float32(B * HW)
    wden = jnp.sum(wc)
    whole = jnp.where(wden > 0.0, jnp.sum(wv) / jnp.maximum(wden, 1.0), 0.0)
    o_ref[...] = jnp.reshape(whole + centroid + tissue, (1, 1))


@jax.jit
def kernel(pred, target):
    pf = pred.reshape(-1)
    tf = target.reshape(-1)
    mesh = plsc.VectorSubcoreMesh(core_axis_name="c", subcore_axis_name="s", num_cores=NC, num_subcores=NS)
    partials = pl.kernel(
        _sc_body,
        out_type=jax.ShapeDtypeStruct((NW, NQ * B, L), jnp.float32),
        mesh=mesh,
        scratch_types=[
            pltpu.VMEM((B, 6, CHUNK), jnp.float32),
            pltpu.VMEM((NQ * B, L), jnp.float32),
        ] + [pltpu.SemaphoreType.DMA] * B,
        compiler_params=pltpu.CompilerParams(use_tc_tiling_on_sc=False),
    )(pf, tf)
    out = pl.pallas_call(
        _combine_body,
        out_shape=jax.ShapeDtypeStruct((1, 1), jnp.float32),
    )(partials)
    return out[0, 0]


# speedup vs baseline: 2.0456x; 1.3271x over previous
import jax
import jax.numpy as jnp
from jax import lax
from jax.experimental import pallas as pl
from jax.experimental.pallas import tpu as pltpu
from jax.experimental.pallas import tpu_sc as plsc

L = 16
NC = 2
NS = 16
NW = NC * NS
B = 8
H = 224
W = 224
HW = H * W
TR = H // 8
UPW = B * TR // NW
NROWVEC = W // L
NQ = 5


def _bce(x, t):
    e = jnp.exp(-jnp.abs(x))
    r = e / (e + 2.0)
    s = r * r
    p = jnp.float32(1.0 / 11.0)
    for c in (1.0 / 9.0, 1.0 / 7.0, 1.0 / 5.0, 1.0 / 3.0, 1.0):
        p = p * s + jnp.float32(c)
    return jnp.maximum(x, 0.0) - x * t + 2.0 * r * p


def _sc_body(pred_hbm, target_hbm, out_hbm, buf, stage, sem0, sem1):
    wid = lax.axis_index("s") * NC + lax.axis_index("c")
    u0 = wid * UPW
    smp = u0 // TR
    row_base = (u0 - smp * TR) * 8
    sems = (sem0, sem1)
    zero = jnp.zeros((L,), jnp.float32)

    def issue(k):
        slot = k % 2
        r0 = row_base + k * 8
        handles = []
        for c in range(3):
            handles.append(pltpu.async_copy(
                pred_hbm.at[smp, c, pl.ds(r0, 8), :], buf.at[slot, c],
                sems[slot]))
            handles.append(pltpu.async_copy(
                target_hbm.at[smp, c, pl.ds(r0, 8), :], buf.at[slot, 3 + c],
                sems[slot]))
        return handles

    inflight = {0: issue(0), 1: issue(1)}
    accs = (zero, zero, zero, zero, zero)

    for k in range(UPW):
        slot = k % 2
        for h in inflight.pop(k):
            h.wait()
        if k + 2 < UPW:
            inflight[k + 2] = issue(k + 2)

        def step(i, carry, slot=slot):
            acc_t2, acc_c, acc_ti, acc_w, acc_wc = carry
            r = i // NROWVEC
            o = (i - r * NROWVEC) * L
            p0 = buf[slot, 0, r, pl.ds(o, L)]
            p1 = buf[slot, 1, r, pl.ds(o, L)]
            p2 = buf[slot, 2, r, pl.ds(o, L)]
            t0 = buf[slot, 3, r, pl.ds(o, L)]
            t1 = buf[slot, 4, r, pl.ds(o, L)]
            t2 = buf[slot, 5, r, pl.ds(o, L)]
            tc = t2 - t2 * t0
            roi = 1.0 - tc * (1.0 - t1)
            wm = jnp.where(roi != 0.0, 1.0, 0.0)
            acc_t2 = acc_t2 + t2
            acc_c = acc_c + _bce(p2, tc)
            acc_ti = acc_ti + _bce(p0, t0)
            acc_w = acc_w + _bce(p1, t1) * wm
            acc_wc = acc_wc + wm
            return acc_t2, acc_c, acc_ti, acc_w, acc_wc

        accs = lax.fori_loop(0, 8 * NROWVEC, step, accs)

    for q in range(NQ):
        stage[q] = accs[q]
    pltpu.sync_copy(stage, out_hbm.at[wid])


def _combine_body(part_ref, o_ref):
    x = part_ref[...]
    t = jnp.sum(jnp.sum(x, axis=1), axis=-1)
    li = lax.broadcasted_iota(jnp.int32, (B, NQ), 1)
    sel = t[:, :1] != 0.0
    zero = jnp.zeros_like(t)
    cnum = jnp.sum(jnp.where((li == 1) & sel, t, zero))
    cden = jnp.sum(jnp.where((li == 1) & sel, jnp.float32(HW), zero))
    tsum = jnp.sum(jnp.where(li == 2, t, zero))
    wsum = jnp.sum(jnp.where(li == 3, t, zero))
    wden = jnp.sum(jnp.where(li == 4, t, zero))
    centroid = jnp.where(cden > 0.0, cnum / jnp.maximum(cden, 1.0), 0.0)
    tissue = tsum / jnp.float32(B * HW)
    whole = jnp.where(wden > 0.0, wsum / jnp.maximum(wden, 1.0), 0.0)
    o_ref[...] = jnp.reshape(whole + centroid + tissue, (1, 1))


@jax.jit
def kernel(pred, target):
    mesh = plsc.VectorSubcoreMesh(core_axis_name="c", subcore_axis_name="s",
                                  num_cores=NC, num_subcores=NS)
    partials = pl.kernel(
        _sc_body,
        out_type=jax.ShapeDtypeStruct((NW, NQ, L), jnp.float32),
        mesh=mesh,
        scratch_types=[
            pltpu.VMEM((2, 6, 8, W), jnp.float32),
            pltpu.VMEM((NQ, L), jnp.float32),
            pltpu.SemaphoreType.DMA,
            pltpu.SemaphoreType.DMA,
        ],
        compiler_params=pltpu.CompilerParams(use_tc_tiling_on_sc=True),
    )(pred, target)
    out = pl.pallas_call(
        _combine_body,
        out_shape=jax.ShapeDtypeStruct((1, 1), jnp.float32),
    )(partials.reshape(B, NW // B, NQ, L))
    return out[0, 0]
